# baseline (speedup 1.0000x reference)
import functools

import jax
import jax.numpy as jnp
from jax import lax
from jax.experimental import pallas as pl
from jax.experimental.pallas import tpu as pltpu
from jax.experimental.pallas import tpu_sc as plsc

N = 10000
E = 320000
HID = 128
NPROT = 4

NC = 2
NS = 16
NW = NC * NS
CHK = 128
NCH = 80
EPAD = NW * NCH * CHK
ACCR = N + 8
ZOFF = 624
ZLEN = 640

ROWS = 1000
GRID = N // ROWS

def _mesh():
    return plsc.VectorSubcoreMesh(core_axis_name="c", subcore_axis_name="s",
                                  num_cores=NC, num_subcores=NS)


def _deg_body(dst3, ones, zrows, out, dst_v, ones_v, acc, sem):
    c = lax.axis_index("c")
    s = lax.axis_index("s")
    wid = c * NS + s
    pltpu.sync_copy(zrows, acc.at[pl.ds(s * ZOFF, ZLEN)])
    pltpu.sync_copy(ones, ones_v)
    pltpu.sync_copy(dst3.at[wid], dst_v)
    plsc.subcore_barrier()

    def chunk(k, carry):
        pltpu.sync_copy(ones_v, acc.at[dst_v.at[k]], add=True)
        return carry

    lax.fori_loop(0, NCH, chunk, 0)
    plsc.subcore_barrier()
    pltpu.sync_copy(acc.at[pl.ds(s * ZOFF, ZLEN)],
                    out.at[pl.ds(c * N + s * ZOFF, ZLEN)])


def _deg_partials(dst3, ones, zrows):
    return pl.kernel(
        _deg_body,
        jax.ShapeDtypeStruct((NC * N, 16), jnp.float32),
        mesh=_mesh(),
        scratch_types=[
            pltpu.VMEM((NCH, CHK), jnp.int32),
            pltpu.VMEM((CHK, 16), jnp.float32),
            pltpu.VMEM_SHARED((ACCR, 16), jnp.float32),
            pltpu.SemaphoreType.DMA,
        ],
    )(dst3, ones, zrows)


def _agg_body(h, src3, dst3, zrows, out, src_v, dst_v, rows_v, acc, sem):
    c = lax.axis_index("c")
    s = lax.axis_index("s")
    wid = c * NS + s
    pltpu.sync_copy(zrows, acc.at[pl.ds(s * ZOFF, ZLEN)])
    pltpu.sync_copy(src3.at[wid], src_v)
    pltpu.sync_copy(dst3.at[wid], dst_v)
    plsc.subcore_barrier()

    def chunk(k, carry):
        pltpu.async_copy(h.at[src_v.at[k]], rows_v, sem).wait()
        pltpu.sync_copy(rows_v, acc.at[dst_v.at[k]], add=True)
        return carry

    lax.fori_loop(0, NCH, chunk, 0)

    plsc.subcore_barrier()
    pltpu.sync_copy(acc.at[pl.ds(s * ZOFF, ZLEN)],
                    out.at[pl.ds(c * N + s * ZOFF, ZLEN)])


def _agg_partials(h, src3, dst3, zrows):
    return pl.kernel(
        _agg_body,
        jax.ShapeDtypeStruct((NC * N, HID), jnp.float32),
        mesh=_mesh(),
        scratch_types=[
            pltpu.VMEM((NCH, CHK), jnp.int32),
            pltpu.VMEM((NCH, CHK), jnp.int32),
            pltpu.VMEM((CHK, HID), jnp.float32),
            pltpu.VMEM_SHARED((ACCR, HID), jnp.float32),
            pltpu.SemaphoreType.DMA,
        ],
    )(h, src3, dst3, zrows)


def _embed_body(x_ref, w_ref, b_ref, o_ref):
    o_ref[...] = lax.dot_general(
        x_ref[...], w_ref[...], (((1,), (1,)), ((), ())),
        preferred_element_type=jnp.float32) + b_ref[...]


def _embed(x, w, b2):
    return pl.pallas_call(
        _embed_body,
        grid=(GRID,),
        in_specs=[
            pl.BlockSpec((ROWS, HID), lambda i: (i, 0)),
            pl.BlockSpec((HID, HID), lambda i: (0, 0)),
            pl.BlockSpec((1, HID), lambda i: (0, 0)),
        ],
        out_specs=pl.BlockSpec((ROWS, HID), lambda i: (i, 0)),
        out_shape=jax.ShapeDtypeStruct((N, HID), jnp.float32),
    )(x, w, b2)


def _layer_body(last, x_ref, ps_ref, dp_ref, w_ref, b_ref, o_ref, *rest):
    x = x_ref[...]
    ps = ps_ref[0] + ps_ref[1]
    deg = dp_ref[0, :, 0:1] + dp_ref[1, :, 0:1]
    agg = ps * (1.0 / jnp.maximum(deg, 1.0))
    w = w_ref[...]
    z = lax.dot_general(x, w[:, :HID], (((1,), (1,)), ((), ())),
                        preferred_element_type=jnp.float32)
    z = z + lax.dot_general(agg, w[:, HID:], (((1,), (1,)), ((), ())),
                            preferred_element_type=jnp.float32)
    z = z + b_ref[...]
    nrm = jnp.sqrt(jnp.sum(z * z, axis=1, keepdims=True))
    z = z / jnp.maximum(nrm, 1e-12)
    o = x + jnp.maximum(z, 0.0)
    o_ref[...] = o
    if last:
        hsum_ref = rest[0]
        @pl.when(pl.program_id(0) == 0)
        def _():
            hsum_ref[...] = jnp.zeros_like(hsum_ref)
        hsum_ref[...] += jnp.sum(o, axis=0, keepdims=True)


def _layer(x, ps, dp, w, b2, last):
    out_shape = [jax.ShapeDtypeStruct((N, HID), jnp.float32)]
    out_specs = [pl.BlockSpec((ROWS, HID), lambda i: (i, 0))]
    if last:
        out_shape.append(jax.ShapeDtypeStruct((1, HID), jnp.float32))
        out_specs.append(pl.BlockSpec((1, HID), lambda i: (0, 0)))
    return pl.pallas_call(
        functools.partial(_layer_body, last),
        grid=(GRID,),
        in_specs=[
            pl.BlockSpec((ROWS, HID), lambda i: (i, 0)),
            pl.BlockSpec((NC, ROWS, HID), lambda i: (0, i, 0)),
            pl.BlockSpec((NC, ROWS, 16), lambda i: (0, i, 0)),
            pl.BlockSpec((HID, 2 * HID), lambda i: (0, 0)),
            pl.BlockSpec((1, HID), lambda i: (0, 0)),
        ],
        out_specs=out_specs,
        out_shape=out_shape,
    )(x, ps, dp, w, b2)


def _head_body(hsum_ref, pp_ref, pn_ref, fc_ref, o_ref):
    hg = hsum_ref[...] * (1.0 / N)
    dp = hg - pp_ref[...]
    dn = hg - pn_ref[...]
    dpos = jnp.sum(dp * dp, axis=1, keepdims=True)
    dneg = jnp.sum(dn * dn, axis=1, keepdims=True)
    spos = jnp.log((dpos + 1.0) / (dpos + 1e-12))
    sneg = jnp.log((dneg + 1.0) / (dneg + 1e-12))
    fc = fc_ref[...]
    y = lax.dot_general(fc[:, :NPROT], spos, (((1,), (0,)), ((), ())),
                        preferred_element_type=jnp.float32)
    y = y + lax.dot_general(fc[:, NPROT:], sneg, (((1,), (0,)), ((), ())),
                            preferred_element_type=jnp.float32)
    o_ref[...] = 1.0 / (1.0 + jnp.exp(-y))


def _head(hsum, pp, pn, fc):
    return pl.pallas_call(
        _head_body,
        out_shape=jax.ShapeDtypeStruct((1, 1), jnp.float32),
    )(hsum, pp, pn, fc)


def kernel(h, edge_index, e, W_embed, b_embed, W0, b0, W1, b1, W2, b2,
           p_pos, p_neg, FC_w):
    pad = EPAD - E
    src3 = jnp.concatenate(
        [edge_index[0], jnp.zeros((pad,), jnp.int32)]).reshape(NW, NCH, CHK)
    dst3 = jnp.concatenate(
        [edge_index[1], jnp.full((pad,), N, jnp.int32)]).reshape(NW, NCH, CHK)
    ones = jnp.ones((CHK, 16), jnp.float32)
    zdeg = jnp.zeros((ZLEN, 16), jnp.float32)
    zrow = jnp.zeros((ZLEN, HID), jnp.float32)

    degp = _deg_partials(dst3, ones, zdeg).reshape(NC, N, 16)
    hcur = _embed(h, W_embed, b_embed.reshape(1, HID))

    for i, (W, b) in enumerate(((W0, b0), (W1, b1), (W2, b2))):
        ps = _agg_partials(hcur, src3, dst3, zrow).reshape(NC, N, HID)
        res = _layer(hcur, ps, degp, W, b.reshape(1, HID), last=(i == 2))
        hcur = res[0]
    hsum = res[1]

    y = _head(hsum, p_pos, p_neg, FC_w)
    return jnp.squeeze(y)


# speedup vs baseline: 1.1762x; 1.1762x over previous
import functools

import jax
import jax.numpy as jnp
from jax import lax
from jax.experimental import pallas as pl
from jax.experimental.pallas import tpu as pltpu
from jax.experimental.pallas import tpu_sc as plsc

N = 10000
E = 320000
HID = 128
NPROT = 4

NC = 2
NS = 16
NW = NC * NS
CHK = 128
NCH = 80
EPW = NCH * CHK
PADW = EPW - E // NW
EPAD = NW * EPW
DEGR = N + 8
ZOFF = 624
ZLEN = 640

ROWS = 1000
GRID = N // ROWS

def _mesh():
    return plsc.VectorSubcoreMesh(core_axis_name="c", subcore_axis_name="s",
                                  num_cores=NC, num_subcores=NS)


def _deg_body(dst3, ones, zrows, out, dst_v, ones_v, acc, sem):
    c = lax.axis_index("c")
    s = lax.axis_index("s")
    wid = c * NS + s
    pltpu.sync_copy(zrows, acc.at[pl.ds(s * ZOFF, ZLEN)])
    pltpu.sync_copy(ones, ones_v)
    pltpu.sync_copy(dst3.at[wid], dst_v)
    plsc.subcore_barrier()

    def chunk(k, carry):
        pltpu.sync_copy(ones_v, acc.at[dst_v.at[k]], add=True)
        return carry

    lax.fori_loop(0, NCH, chunk, 0)
    plsc.subcore_barrier()
    pltpu.sync_copy(acc.at[pl.ds(s * ZOFF, ZLEN)],
                    out.at[pl.ds(c * N + s * ZOFF, ZLEN)])


def _deg_partials(dst3, ones, zrows):
    return pl.kernel(
        _deg_body,
        jax.ShapeDtypeStruct((NC * N, 16), jnp.float32),
        mesh=_mesh(),
        scratch_types=[
            pltpu.VMEM((NCH, CHK), jnp.int32),
            pltpu.VMEM((CHK, 16), jnp.float32),
            pltpu.VMEM_SHARED((DEGR, 16), jnp.float32),
            pltpu.SemaphoreType.DMA,
        ],
    )(dst3, ones, zrows)


def _agg_body(h, src3, dst3, zrows, out, src_v, dst_v, rows_v, acc, sem):
    c = lax.axis_index("c")
    s = lax.axis_index("s")
    wid = c * NS + s
    pltpu.sync_copy(zrows, acc.at[pl.ds(s * ZOFF, ZLEN)])
    pltpu.sync_copy(src3.at[wid], src_v)
    pltpu.sync_copy(dst3.at[wid], dst_v)
    plsc.subcore_barrier()

    def chunk(k, carry):
        pltpu.async_copy(h.at[src_v.at[k]], rows_v, sem).wait()
        pltpu.sync_copy(rows_v, acc.at[dst_v.at[k]], add=True)
        return carry

    lax.fori_loop(0, NCH, chunk, 0)

    plsc.subcore_barrier()
    pltpu.sync_copy(acc.at[pl.ds(s * ZOFF, ZLEN)],
                    out.at[pl.ds(c * N + s * ZOFF, ZLEN)])


def _agg_partials(h, src3, dst3, zrows):
    return pl.kernel(
        _agg_body,
        jax.ShapeDtypeStruct((NC * N, HID), jnp.float32),
        mesh=_mesh(),
        scratch_types=[
            pltpu.VMEM((NCH, CHK), jnp.int32),
            pltpu.VMEM((NCH, CHK), jnp.int32),
            pltpu.VMEM((CHK, HID), jnp.float32),
            pltpu.VMEM_SHARED((N, HID), jnp.float32),
            pltpu.SemaphoreType.DMA,
        ],
    )(h, src3, dst3, zrows)


def _embed_body(x_ref, w_ref, b_ref, o_ref):
    o_ref[...] = lax.dot_general(
        x_ref[...], w_ref[...], (((1,), (1,)), ((), ())),
        preferred_element_type=jnp.float32) + b_ref[...]


def _embed(x, w, b2):
    return pl.pallas_call(
        _embed_body,
        grid=(GRID,),
        in_specs=[
            pl.BlockSpec((ROWS, HID), lambda i: (i, 0)),
            pl.BlockSpec((HID, HID), lambda i: (0, 0)),
            pl.BlockSpec((1, HID), lambda i: (0, 0)),
        ],
        out_specs=pl.BlockSpec((ROWS, HID), lambda i: (i, 0)),
        out_shape=jax.ShapeDtypeStruct((N, HID), jnp.float32),
    )(x, w, b2)


def _layer_body(last, x_ref, ps_ref, dp_ref, w_ref, b_ref, o_ref, *rest):
    x = x_ref[...]
    ps = ps_ref[0] + ps_ref[1]
    deg = dp_ref[0, :, 0:1] + dp_ref[1, :, 0:1]
    agg = ps * (1.0 / jnp.maximum(deg, 1.0))
    w = w_ref[...]
    z = lax.dot_general(x, w[:, :HID], (((1,), (1,)), ((), ())),
                        preferred_element_type=jnp.float32)
    z = z + lax.dot_general(agg, w[:, HID:], (((1,), (1,)), ((), ())),
                            preferred_element_type=jnp.float32)
    z = z + b_ref[...]
    nrm = jnp.sqrt(jnp.sum(z * z, axis=1, keepdims=True))
    z = z / jnp.maximum(nrm, 1e-12)
    o = x + jnp.maximum(z, 0.0)
    o_ref[...] = o
    if last:
        hsum_ref = rest[0]
        @pl.when(pl.program_id(0) == 0)
        def _():
            hsum_ref[...] = jnp.zeros_like(hsum_ref)
        hsum_ref[...] += jnp.sum(o, axis=0, keepdims=True)


def _layer(x, ps, dp, w, b2, last):
    out_shape = [jax.ShapeDtypeStruct((N, HID), jnp.float32)]
    out_specs = [pl.BlockSpec((ROWS, HID), lambda i: (i, 0))]
    if last:
        out_shape.append(jax.ShapeDtypeStruct((1, HID), jnp.float32))
        out_specs.append(pl.BlockSpec((1, HID), lambda i: (0, 0)))
    return pl.pallas_call(
        functools.partial(_layer_body, last),
        grid=(GRID,),
        in_specs=[
            pl.BlockSpec((ROWS, HID), lambda i: (i, 0)),
            pl.BlockSpec((NC, ROWS, HID), lambda i: (0, i, 0)),
            pl.BlockSpec((NC, ROWS, 16), lambda i: (0, i, 0)),
            pl.BlockSpec((HID, 2 * HID), lambda i: (0, 0)),
            pl.BlockSpec((1, HID), lambda i: (0, 0)),
        ],
        out_specs=out_specs,
        out_shape=out_shape,
    )(x, ps, dp, w, b2)


def _head_body(hsum_ref, pp_ref, pn_ref, fc_ref, o_ref):
    hg = hsum_ref[...] * (1.0 / N)
    dp = hg - pp_ref[...]
    dn = hg - pn_ref[...]
    dpos = jnp.sum(dp * dp, axis=1, keepdims=True)
    dneg = jnp.sum(dn * dn, axis=1, keepdims=True)
    spos = jnp.log((dpos + 1.0) / (dpos + 1e-12))
    sneg = jnp.log((dneg + 1.0) / (dneg + 1e-12))
    fc = fc_ref[...]
    y = lax.dot_general(fc[:, :NPROT], spos, (((1,), (0,)), ((), ())),
                        preferred_element_type=jnp.float32)
    y = y + lax.dot_general(fc[:, NPROT:], sneg, (((1,), (0,)), ((), ())),
                            preferred_element_type=jnp.float32)
    o_ref[...] = 1.0 / (1.0 + jnp.exp(-y))


def _head(hsum, pp, pn, fc):
    return pl.pallas_call(
        _head_body,
        out_shape=jax.ShapeDtypeStruct((1, 1), jnp.float32),
    )(hsum, pp, pn, fc)


def kernel(h, edge_index, e, W_embed, b_embed, W0, b0, W1, b1, W2, b2,
           p_pos, p_neg, FC_w):
    src2 = edge_index[0].reshape(NW, E // NW)
    dst2 = edge_index[1].reshape(NW, E // NW)
    psrc = jnp.full((NW, PADW), N, jnp.int32)
    pdst_agg = jnp.broadcast_to(
        (jnp.arange(PADW, dtype=jnp.int32) * 41) % N, (NW, PADW))
    pdst_deg = jnp.full((NW, PADW), N, jnp.int32)
    src3 = jnp.concatenate([src2, psrc], axis=1).reshape(NW, NCH, CHK)
    dst3 = jnp.concatenate([dst2, pdst_agg], axis=1).reshape(NW, NCH, CHK)
    dstd3 = jnp.concatenate([dst2, pdst_deg], axis=1).reshape(NW, NCH, CHK)
    ones = jnp.ones((CHK, 16), jnp.float32)
    zdeg = jnp.zeros((ZLEN, 16), jnp.float32)
    zrow = jnp.zeros((ZLEN, HID), jnp.float32)

    degp = _deg_partials(dstd3, ones, zdeg).reshape(NC, N, 16)
    hcur = _embed(h, W_embed, b_embed.reshape(1, HID))

    for i, (W, b) in enumerate(((W0, b0), (W1, b1), (W2, b2))):
        hpad = jnp.concatenate([hcur, zrow[:8]], axis=0)
        ps = _agg_partials(hpad, src3, dst3, zrow).reshape(NC, N, HID)
        res = _layer(hcur, ps, degp, W, b.reshape(1, HID), last=(i == 2))
        hcur = res[0]
    hsum = res[1]

    y = _head(hsum, p_pos, p_neg, FC_w)
    return jnp.squeeze(y)


# speedup vs baseline: 2.3851x; 2.0277x over previous
import functools

import jax
import jax.numpy as jnp
from jax import lax
from jax.experimental import pallas as pl
from jax.experimental.pallas import tpu as pltpu
from jax.experimental.pallas import tpu_sc as plsc

N = 10000
E = 320000
HID = 128
NPROT = 4

NC = 2
NS = 16
NW = NC * NS
CHK = 80
NCH = (E // NW) // CHK
DEGR = N
ZOFF = 624
ZLEN = 640

ROWS = 1000
GRID = N // ROWS

def _mesh():
    return plsc.VectorSubcoreMesh(core_axis_name="c", subcore_axis_name="s",
                                  num_cores=NC, num_subcores=NS)


def _deg_body(dst3, ones, zrows, out, dst_v, ones_v, acc, sem):
    c = lax.axis_index("c")
    s = lax.axis_index("s")
    wid = c * NS + s
    pltpu.sync_copy(zrows, acc.at[pl.ds(s * ZOFF, ZLEN)])
    pltpu.sync_copy(ones, ones_v)
    pltpu.sync_copy(dst3.at[wid], dst_v)
    plsc.subcore_barrier()

    def chunk(k, carry):
        pltpu.sync_copy(ones_v, acc.at[dst_v.at[k]], add=True)
        return carry

    lax.fori_loop(0, NCH, chunk, 0)
    plsc.subcore_barrier()
    pltpu.sync_copy(acc.at[pl.ds(s * ZOFF, ZLEN)],
                    out.at[pl.ds(c * N + s * ZOFF, ZLEN)])


def _deg_partials(dst3, ones, zrows):
    return pl.kernel(
        _deg_body,
        jax.ShapeDtypeStruct((NC * N, 16), jnp.float32),
        mesh=_mesh(),
        scratch_types=[
            pltpu.VMEM((NCH, CHK), jnp.int32),
            pltpu.VMEM((CHK, 16), jnp.float32),
            pltpu.VMEM_SHARED((DEGR, 16), jnp.float32),
            pltpu.SemaphoreType.DMA,
        ],
    )(dst3, ones, zrows)


def _agg_body(h, src3, dst3, zrows, out, src_v, dst_v, rows_v, acc, sem):
    c = lax.axis_index("c")
    s = lax.axis_index("s")
    wid = c * NS + s
    pltpu.sync_copy(zrows, acc.at[pl.ds(s * ZOFF, ZLEN)])
    pltpu.sync_copy(src3.at[wid], src_v)
    pltpu.sync_copy(dst3.at[wid], dst_v)
    plsc.subcore_barrier()

    def chunk(k, carry):
        pltpu.async_copy(h.at[src_v.at[k]], rows_v, sem).wait()
        pltpu.sync_copy(rows_v, acc.at[dst_v.at[k]], add=True)
        return carry

    lax.fori_loop(0, NCH, chunk, 0)

    plsc.subcore_barrier()
    pltpu.sync_copy(acc.at[pl.ds(s * ZOFF, ZLEN)],
                    out.at[pl.ds(c * N + s * ZOFF, ZLEN)])


def _agg_partials(h, src3, dst3, zrows):
    return pl.kernel(
        _agg_body,
        jax.ShapeDtypeStruct((NC * N, HID), jnp.float32),
        mesh=_mesh(),
        scratch_types=[
            pltpu.VMEM((NCH, CHK), jnp.int32),
            pltpu.VMEM((NCH, CHK), jnp.int32),
            pltpu.VMEM((CHK, HID), jnp.float32),
            pltpu.VMEM_SHARED((N, HID), jnp.float32),
            pltpu.SemaphoreType.DMA,
        ],
    )(h, src3, dst3, zrows)


def _embed_body(x_ref, w_ref, b_ref, o_ref):
    o_ref[...] = lax.dot_general(
        x_ref[...], w_ref[...], (((1,), (1,)), ((), ())),
        preferred_element_type=jnp.float32) + b_ref[...]


def _embed(x, w, b2):
    return pl.pallas_call(
        _embed_body,
        grid=(GRID,),
        in_specs=[
            pl.BlockSpec((ROWS, HID), lambda i: (i, 0)),
            pl.BlockSpec((HID, HID), lambda i: (0, 0)),
            pl.BlockSpec((1, HID), lambda i: (0, 0)),
        ],
        out_specs=pl.BlockSpec((ROWS, HID), lambda i: (i, 0)),
        out_shape=jax.ShapeDtypeStruct((N, HID), jnp.float32),
    )(x, w, b2)


def _layer_body(last, x_ref, ps_ref, dp_ref, w_ref, b_ref, o_ref, *rest):
    x = x_ref[...]
    ps = ps_ref[0] + ps_ref[1]
    deg = dp_ref[0, :, 0:1] + dp_ref[1, :, 0:1]
    agg = ps * (1.0 / jnp.maximum(deg, 1.0))
    w = w_ref[...]
    z = lax.dot_general(x, w[:, :HID], (((1,), (1,)), ((), ())),
                        preferred_element_type=jnp.float32)
    z = z + lax.dot_general(agg, w[:, HID:], (((1,), (1,)), ((), ())),
                            preferred_element_type=jnp.float32)
    z = z + b_ref[...]
    nrm = jnp.sqrt(jnp.sum(z * z, axis=1, keepdims=True))
    z = z / jnp.maximum(nrm, 1e-12)
    o = x + jnp.maximum(z, 0.0)
    o_ref[...] = o
    if last:
        hsum_ref = rest[0]
        @pl.when(pl.program_id(0) == 0)
        def _():
            hsum_ref[...] = jnp.zeros_like(hsum_ref)
        hsum_ref[...] += jnp.sum(o, axis=0, keepdims=True)


def _layer(x, ps, dp, w, b2, last):
    out_shape = [jax.ShapeDtypeStruct((N, HID), jnp.float32)]
    out_specs = [pl.BlockSpec((ROWS, HID), lambda i: (i, 0))]
    if last:
        out_shape.append(jax.ShapeDtypeStruct((1, HID), jnp.float32))
        out_specs.append(pl.BlockSpec((1, HID), lambda i: (0, 0)))
    return pl.pallas_call(
        functools.partial(_layer_body, last),
        grid=(GRID,),
        in_specs=[
            pl.BlockSpec((ROWS, HID), lambda i: (i, 0)),
            pl.BlockSpec((NC, ROWS, HID), lambda i: (0, i, 0)),
            pl.BlockSpec((NC, ROWS, 16), lambda i: (0, i, 0)),
            pl.BlockSpec((HID, 2 * HID), lambda i: (0, 0)),
            pl.BlockSpec((1, HID), lambda i: (0, 0)),
        ],
        out_specs=out_specs,
        out_shape=out_shape,
    )(x, ps, dp, w, b2)


def _head_body(hsum_ref, pp_ref, pn_ref, fc_ref, o_ref):
    hg = hsum_ref[...] * (1.0 / N)
    dp = hg - pp_ref[...]
    dn = hg - pn_ref[...]
    dpos = jnp.sum(dp * dp, axis=1, keepdims=True)
    dneg = jnp.sum(dn * dn, axis=1, keepdims=True)
    spos = jnp.log((dpos + 1.0) / (dpos + 1e-12))
    sneg = jnp.log((dneg + 1.0) / (dneg + 1e-12))
    fc = fc_ref[...]
    y = lax.dot_general(fc[:, :NPROT], spos, (((1,), (0,)), ((), ())),
                        preferred_element_type=jnp.float32)
    y = y + lax.dot_general(fc[:, NPROT:], sneg, (((1,), (0,)), ((), ())),
                            preferred_element_type=jnp.float32)
    o_ref[...] = 1.0 / (1.0 + jnp.exp(-y))


def _head(hsum, pp, pn, fc):
    return pl.pallas_call(
        _head_body,
        out_shape=jax.ShapeDtypeStruct((1, 1), jnp.float32),
    )(hsum, pp, pn, fc)


def kernel(h, edge_index, e, W_embed, b_embed, W0, b0, W1, b1, W2, b2,
           p_pos, p_neg, FC_w):
    src3 = edge_index[0].reshape(NW, NCH, CHK)
    dst3 = edge_index[1].reshape(NW, NCH, CHK)
    ones = jnp.ones((CHK, 16), jnp.float32)
    zdeg = jnp.zeros((ZLEN, 16), jnp.float32)
    zrow = jnp.zeros((ZLEN, HID), jnp.float32)

    degp = _deg_partials(dst3, ones, zdeg).reshape(NC, N, 16)
    hcur = _embed(h, W_embed, b_embed.reshape(1, HID))

    for i, (W, b) in enumerate(((W0, b0), (W1, b1), (W2, b2))):
        ps = _agg_partials(hcur, src3, dst3, zrow).reshape(NC, N, HID)
        res = _layer(hcur, ps, degp, W, b.reshape(1, HID), last=(i == 2))
        hcur = res[0]
    hsum = res[1]

    y = _head(hsum, p_pos, p_neg, FC_w)
    return jnp.squeeze(y)


# speedup vs baseline: 2.9118x; 1.2208x over previous
import functools

import jax
import jax.numpy as jnp
from jax import lax
from jax.experimental import pallas as pl
from jax.experimental.pallas import tpu as pltpu
from jax.experimental.pallas import tpu_sc as plsc

N = 10000
E = 320000
HID = 128
NPROT = 4

NC = 2
NS = 16
NW = NC * NS
CHK = 80
NCH = (E // NW) // CHK
DEGR = N
ZOFF = 624
ZLEN = 640

ROWS = 1000
GRID = N // ROWS

def _mesh():
    return plsc.VectorSubcoreMesh(core_axis_name="c", subcore_axis_name="s",
                                  num_cores=NC, num_subcores=NS)


def _deg_body(dst3, ones, zrows, out, dst_v, ones_v, acc, sem):
    c = lax.axis_index("c")
    s = lax.axis_index("s")
    wid = c * NS + s
    pltpu.sync_copy(zrows, acc.at[pl.ds(s * ZOFF, ZLEN)])
    pltpu.sync_copy(ones, ones_v)
    pltpu.sync_copy(dst3.at[wid], dst_v)
    plsc.subcore_barrier()

    def chunk(k, carry):
        pltpu.sync_copy(ones_v, acc.at[dst_v.at[k]], add=True)
        return carry

    lax.fori_loop(0, NCH, chunk, 0)
    plsc.subcore_barrier()
    pltpu.sync_copy(acc.at[pl.ds(s * ZOFF, ZLEN)],
                    out.at[pl.ds(c * N + s * ZOFF, ZLEN)])


def _deg_partials(dst3, ones, zrows):
    return pl.kernel(
        _deg_body,
        jax.ShapeDtypeStruct((NC * N, 16), jnp.float32),
        mesh=_mesh(),
        scratch_types=[
            pltpu.VMEM((NCH, CHK), jnp.int32),
            pltpu.VMEM((CHK, 16), jnp.float32),
            pltpu.VMEM_SHARED((DEGR, 16), jnp.float32),
            pltpu.SemaphoreType.DMA,
        ],
    )(dst3, ones, zrows)


HA = 63
HB = NCH - HA


def _agg_body(h, srcA3, srcB3, dst3, zrows, out,
              src_v, dst_v, rows0, rows1, acc, sem0, sem1):
    c = lax.axis_index("c")
    s = lax.axis_index("s")
    wid = c * NS + s
    pltpu.sync_copy(zrows, acc.at[pl.ds(s * ZOFF, ZLEN)])
    pltpu.sync_copy(srcA3.at[wid], src_v)
    pltpu.sync_copy(dst3.at[wid], dst_v)
    plsc.subcore_barrier()

    bufs = (rows0, rows1)
    sems = (sem0, sem1)

    def fire(kl, b):
        return pltpu.async_copy(h.at[src_v.at[kl]], bufs[b], sems[b])

    def scat(kg, b):
        pltpu.sync_copy(bufs[b], acc.at[dst_v.at[kg]], add=True)

    def pair_a(j, carry):
        k = 2 * j
        cp0 = fire(k, 0)
        cp1 = fire(k + 1, 1)
        cp0.wait()
        scat(k, 0)
        cp1.wait()
        scat(k + 1, 1)
        return carry

    lax.fori_loop(0, HA // 2, pair_a, 0)
    cp = fire(HA - 1, 0)
    cp.wait()
    scat(HA - 1, 0)
    pltpu.sync_copy(srcB3.at[wid], src_v.at[pl.ds(0, HB)])

    def pair_b(j, carry):
        k = 2 * j
        cp0 = fire(k, 0)
        cp1 = fire(k + 1, 1)
        cp0.wait()
        scat(HA + k, 0)
        cp1.wait()
        scat(HA + k + 1, 1)
        return carry

    lax.fori_loop(0, HB // 2, pair_b, 0)

    plsc.subcore_barrier()
    pltpu.sync_copy(acc.at[pl.ds(s * ZOFF, ZLEN)],
                    out.at[pl.ds(c * N + s * ZOFF, ZLEN)])


def _agg_partials(h, src3, dst3, zrows):
    return pl.kernel(
        _agg_body,
        jax.ShapeDtypeStruct((NC * N, HID), jnp.float32),
        mesh=_mesh(),
        scratch_types=[
            pltpu.VMEM((HA, CHK), jnp.int32),
            pltpu.VMEM((NCH, CHK), jnp.int32),
            pltpu.VMEM((CHK, HID), jnp.float32),
            pltpu.VMEM((CHK, HID), jnp.float32),
            pltpu.VMEM_SHARED((N, HID), jnp.float32),
            pltpu.SemaphoreType.DMA,
            pltpu.SemaphoreType.DMA,
        ],
    )(h, src3[:, :HA], src3[:, HA:], dst3, zrows)


def _embed_body(x_ref, w_ref, b_ref, o_ref):
    o_ref[...] = lax.dot_general(
        x_ref[...], w_ref[...], (((1,), (1,)), ((), ())),
        preferred_element_type=jnp.float32) + b_ref[...]


def _embed(x, w, b2):
    return pl.pallas_call(
        _embed_body,
        grid=(GRID,),
        in_specs=[
            pl.BlockSpec((ROWS, HID), lambda i: (i, 0)),
            pl.BlockSpec((HID, HID), lambda i: (0, 0)),
            pl.BlockSpec((1, HID), lambda i: (0, 0)),
        ],
        out_specs=pl.BlockSpec((ROWS, HID), lambda i: (i, 0)),
        out_shape=jax.ShapeDtypeStruct((N, HID), jnp.float32),
    )(x, w, b2)


def _layer_body(last, x_ref, ps_ref, dp_ref, w_ref, b_ref, o_ref, *rest):
    x = x_ref[...]
    ps = ps_ref[0] + ps_ref[1]
    deg = dp_ref[0, :, 0:1] + dp_ref[1, :, 0:1]
    agg = ps * (1.0 / jnp.maximum(deg, 1.0))
    w = w_ref[...]
    z = lax.dot_general(x, w[:, :HID], (((1,), (1,)), ((), ())),
                        preferred_element_type=jnp.float32)
    z = z + lax.dot_general(agg, w[:, HID:], (((1,), (1,)), ((), ())),
                            preferred_element_type=jnp.float32)
    z = z + b_ref[...]
    nrm = jnp.sqrt(jnp.sum(z * z, axis=1, keepdims=True))
    z = z / jnp.maximum(nrm, 1e-12)
    o = x + jnp.maximum(z, 0.0)
    o_ref[...] = o
    if last:
        hsum_ref = rest[0]
        @pl.when(pl.program_id(0) == 0)
        def _():
            hsum_ref[...] = jnp.zeros_like(hsum_ref)
        hsum_ref[...] += jnp.sum(o, axis=0, keepdims=True)


def _layer(x, ps, dp, w, b2, last):
    out_shape = [jax.ShapeDtypeStruct((N, HID), jnp.float32)]
    out_specs = [pl.BlockSpec((ROWS, HID), lambda i: (i, 0))]
    if last:
        out_shape.append(jax.ShapeDtypeStruct((1, HID), jnp.float32))
        out_specs.append(pl.BlockSpec((1, HID), lambda i: (0, 0)))
    return pl.pallas_call(
        functools.partial(_layer_body, last),
        grid=(GRID,),
        in_specs=[
            pl.BlockSpec((ROWS, HID), lambda i: (i, 0)),
            pl.BlockSpec((NC, ROWS, HID), lambda i: (0, i, 0)),
            pl.BlockSpec((NC, ROWS, 16), lambda i: (0, i, 0)),
            pl.BlockSpec((HID, 2 * HID), lambda i: (0, 0)),
            pl.BlockSpec((1, HID), lambda i: (0, 0)),
        ],
        out_specs=out_specs,
        out_shape=out_shape,
    )(x, ps, dp, w, b2)


def _head_body(hsum_ref, pp_ref, pn_ref, fc_ref, o_ref):
    hg = hsum_ref[...] * (1.0 / N)
    dp = hg - pp_ref[...]
    dn = hg - pn_ref[...]
    dpos = jnp.sum(dp * dp, axis=1, keepdims=True)
    dneg = jnp.sum(dn * dn, axis=1, keepdims=True)
    spos = jnp.log((dpos + 1.0) / (dpos + 1e-12))
    sneg = jnp.log((dneg + 1.0) / (dneg + 1e-12))
    fc = fc_ref[...]
    y = lax.dot_general(fc[:, :NPROT], spos, (((1,), (0,)), ((), ())),
                        preferred_element_type=jnp.float32)
    y = y + lax.dot_general(fc[:, NPROT:], sneg, (((1,), (0,)), ((), ())),
                            preferred_element_type=jnp.float32)
    o_ref[...] = 1.0 / (1.0 + jnp.exp(-y))


def _head(hsum, pp, pn, fc):
    return pl.pallas_call(
        _head_body,
        out_shape=jax.ShapeDtypeStruct((1, 1), jnp.float32),
    )(hsum, pp, pn, fc)


def kernel(h, edge_index, e, W_embed, b_embed, W0, b0, W1, b1, W2, b2,
           p_pos, p_neg, FC_w):
    src3 = edge_index[0].reshape(NW, NCH, CHK)
    dst3 = edge_index[1].reshape(NW, NCH, CHK)
    ones = jnp.ones((CHK, 16), jnp.float32)
    zdeg = jnp.zeros((ZLEN, 16), jnp.float32)
    zrow = jnp.zeros((ZLEN, HID), jnp.float32)

    degp = _deg_partials(dst3, ones, zdeg).reshape(NC, N, 16)
    hcur = _embed(h, W_embed, b_embed.reshape(1, HID))

    for i, (W, b) in enumerate(((W0, b0), (W1, b1), (W2, b2))):
        ps = _agg_partials(hcur, src3, dst3, zrow).reshape(NC, N, HID)
        res = _layer(hcur, ps, degp, W, b.reshape(1, HID), last=(i == 2))
        hcur = res[0]
    hsum = res[1]

    y = _head(hsum, p_pos, p_neg, FC_w)
    return jnp.squeeze(y)


# speedup vs baseline: 3.1878x; 1.0948x over previous
import functools

import jax
import jax.numpy as jnp
from jax import lax
from jax.experimental import pallas as pl
from jax.experimental.pallas import tpu as pltpu
from jax.experimental.pallas import tpu_sc as plsc

N = 10000
E = 320000
HID = 128
NPROT = 4

NC = 2
NS = 16
NW = NC * NS
CHK = 80
NCH = (E // NW) // CHK
DEGR = N
ZOFF = 624
ZLEN = 640

ROWS = 1000
GRID = N // ROWS

def _mesh():
    return plsc.VectorSubcoreMesh(core_axis_name="c", subcore_axis_name="s",
                                  num_cores=NC, num_subcores=NS)


def _deg_body(dst3, ones, zrows, out, dst_v, ones_v, acc, sem):
    c = lax.axis_index("c")
    s = lax.axis_index("s")
    wid = c * NS + s
    pltpu.sync_copy(zrows, acc.at[pl.ds(s * ZOFF, ZLEN)])
    pltpu.sync_copy(ones, ones_v)
    pltpu.sync_copy(dst3.at[wid], dst_v)
    plsc.subcore_barrier()

    def chunk(k, carry):
        pltpu.sync_copy(ones_v, acc.at[dst_v.at[k]], add=True)
        return carry

    lax.fori_loop(0, NCH, chunk, 0)
    plsc.subcore_barrier()
    pltpu.sync_copy(acc.at[pl.ds(s * ZOFF, ZLEN)],
                    out.at[pl.ds(c * N + s * ZOFF, ZLEN)])


def _deg_partials(dst3, ones, zrows):
    return pl.kernel(
        _deg_body,
        jax.ShapeDtypeStruct((NC * N, 16), jnp.float32),
        mesh=_mesh(),
        scratch_types=[
            pltpu.VMEM((NCH, CHK), jnp.int32),
            pltpu.VMEM((CHK, 16), jnp.float32),
            pltpu.VMEM_SHARED((DEGR, 16), jnp.float32),
            pltpu.SemaphoreType.DMA,
        ],
    )(dst3, ones, zrows)


HA = 63
HB = NCH - HA


def _agg_body(h, srcA3, srcB3, dst3, zrows, out,
              src_v, dst_v, rows0, rows1, acc, sem0, sem1):
    c = lax.axis_index("c")
    s = lax.axis_index("s")
    wid = c * NS + s
    pltpu.sync_copy(zrows, acc.at[pl.ds(s * ZOFF, ZLEN)])
    pltpu.sync_copy(srcA3.at[wid], src_v)
    pltpu.sync_copy(dst3.at[wid], dst_v)
    plsc.subcore_barrier()

    bufs = (rows0, rows1)
    sems = (sem0, sem1)

    def fire(kl, b):
        return pltpu.async_copy(h.at[src_v.at[kl]], bufs[b], sems[b])

    def scat(kg, b):
        pltpu.sync_copy(bufs[b], acc.at[dst_v.at[kg]], add=True)

    def quad(k, off):
        cp0 = fire(k, 0)
        cp1 = fire(k + 1, 1)
        cp0.wait()
        scat(off + k, 0)
        cp2 = fire(k + 2, 0)
        cp1.wait()
        scat(off + k + 1, 1)
        cp3 = fire(k + 3, 1)
        cp2.wait()
        scat(off + k + 2, 0)
        cp3.wait()
        scat(off + k + 3, 1)

    def pair(k, off):
        cp0 = fire(k, 0)
        cp1 = fire(k + 1, 1)
        cp0.wait()
        scat(off + k, 0)
        cp1.wait()
        scat(off + k + 1, 1)

    def quad_a(j, carry):
        quad(4 * j, 0)
        return carry

    lax.fori_loop(0, 15, quad_a, 0)
    pair(60, 0)
    cp = fire(62, 0)
    cp.wait()
    scat(62, 0)
    pltpu.sync_copy(srcB3.at[wid], src_v.at[pl.ds(0, HB)])

    def quad_b(j, carry):
        quad(4 * j, HA)
        return carry

    lax.fori_loop(0, 15, quad_b, 0)
    pair(60, HA)

    plsc.subcore_barrier()
    pltpu.sync_copy(acc.at[pl.ds(s * ZOFF, ZLEN)],
                    out.at[pl.ds(c * N + s * ZOFF, ZLEN)])


def _agg_partials(h, src3, dst3, zrows):
    return pl.kernel(
        _agg_body,
        jax.ShapeDtypeStruct((NC * N, HID), jnp.float32),
        mesh=_mesh(),
        scratch_types=[
            pltpu.VMEM((HA, CHK), jnp.int32),
            pltpu.VMEM((NCH, CHK), jnp.int32),
            pltpu.VMEM((CHK, HID), jnp.float32),
            pltpu.VMEM((CHK, HID), jnp.float32),
            pltpu.VMEM_SHARED((N, HID), jnp.float32),
            pltpu.SemaphoreType.DMA,
            pltpu.SemaphoreType.DMA,
        ],
    )(h, src3[:, :HA], src3[:, HA:], dst3, zrows)


def _embed_body(x_ref, w_ref, b_ref, o_ref):
    o_ref[...] = lax.dot_general(
        x_ref[...], w_ref[...], (((1,), (1,)), ((), ())),
        preferred_element_type=jnp.float32) + b_ref[...]


def _embed(x, w, b2):
    return pl.pallas_call(
        _embed_body,
        grid=(GRID,),
        in_specs=[
            pl.BlockSpec((ROWS, HID), lambda i: (i, 0)),
            pl.BlockSpec((HID, HID), lambda i: (0, 0)),
            pl.BlockSpec((1, HID), lambda i: (0, 0)),
        ],
        out_specs=pl.BlockSpec((ROWS, HID), lambda i: (i, 0)),
        out_shape=jax.ShapeDtypeStruct((N, HID), jnp.float32),
    )(x, w, b2)


def _layer_body(last, x_ref, ps_ref, dp_ref, w_ref, b_ref, o_ref, *rest):
    x = x_ref[...]
    ps = ps_ref[0] + ps_ref[1]
    deg = dp_ref[0, :, 0:1] + dp_ref[1, :, 0:1]
    agg = ps * (1.0 / jnp.maximum(deg, 1.0))
    w = w_ref[...]
    z = lax.dot_general(x, w[:, :HID], (((1,), (1,)), ((), ())),
                        preferred_element_type=jnp.float32)
    z = z + lax.dot_general(agg, w[:, HID:], (((1,), (1,)), ((), ())),
                            preferred_element_type=jnp.float32)
    z = z + b_ref[...]
    nrm = jnp.sqrt(jnp.sum(z * z, axis=1, keepdims=True))
    z = z / jnp.maximum(nrm, 1e-12)
    o = x + jnp.maximum(z, 0.0)
    o_ref[...] = o
    if last:
        hsum_ref = rest[0]
        @pl.when(pl.program_id(0) == 0)
        def _():
            hsum_ref[...] = jnp.zeros_like(hsum_ref)
        hsum_ref[...] += jnp.sum(o, axis=0, keepdims=True)


def _layer(x, ps, dp, w, b2, last):
    out_shape = [jax.ShapeDtypeStruct((N, HID), jnp.float32)]
    out_specs = [pl.BlockSpec((ROWS, HID), lambda i: (i, 0))]
    if last:
        out_shape.append(jax.ShapeDtypeStruct((1, HID), jnp.float32))
        out_specs.append(pl.BlockSpec((1, HID), lambda i: (0, 0)))
    return pl.pallas_call(
        functools.partial(_layer_body, last),
        grid=(GRID,),
        in_specs=[
            pl.BlockSpec((ROWS, HID), lambda i: (i, 0)),
            pl.BlockSpec((NC, ROWS, HID), lambda i: (0, i, 0)),
            pl.BlockSpec((NC, ROWS, 16), lambda i: (0, i, 0)),
            pl.BlockSpec((HID, 2 * HID), lambda i: (0, 0)),
            pl.BlockSpec((1, HID), lambda i: (0, 0)),
        ],
        out_specs=out_specs,
        out_shape=out_shape,
    )(x, ps, dp, w, b2)


def _head_body(hsum_ref, pp_ref, pn_ref, fc_ref, o_ref):
    hg = hsum_ref[...] * (1.0 / N)
    dp = hg - pp_ref[...]
    dn = hg - pn_ref[...]
    dpos = jnp.sum(dp * dp, axis=1, keepdims=True)
    dneg = jnp.sum(dn * dn, axis=1, keepdims=True)
    spos = jnp.log((dpos + 1.0) / (dpos + 1e-12))
    sneg = jnp.log((dneg + 1.0) / (dneg + 1e-12))
    fc = fc_ref[...]
    y = lax.dot_general(fc[:, :NPROT], spos, (((1,), (0,)), ((), ())),
                        preferred_element_type=jnp.float32)
    y = y + lax.dot_general(fc[:, NPROT:], sneg, (((1,), (0,)), ((), ())),
                            preferred_element_type=jnp.float32)
    o_ref[...] = 1.0 / (1.0 + jnp.exp(-y))


def _head(hsum, pp, pn, fc):
    return pl.pallas_call(
        _head_body,
        out_shape=jax.ShapeDtypeStruct((1, 1), jnp.float32),
    )(hsum, pp, pn, fc)


def kernel(h, edge_index, e, W_embed, b_embed, W0, b0, W1, b1, W2, b2,
           p_pos, p_neg, FC_w):
    src3 = edge_index[0].reshape(NW, NCH, CHK)
    dst3 = edge_index[1].reshape(NW, NCH, CHK)
    ones = jnp.ones((CHK, 16), jnp.float32)
    zdeg = jnp.zeros((ZLEN, 16), jnp.float32)
    zrow = jnp.zeros((ZLEN, HID), jnp.float32)

    degp = _deg_partials(dst3, ones, zdeg).reshape(NC, N, 16)
    hcur = _embed(h, W_embed, b_embed.reshape(1, HID))

    for i, (W, b) in enumerate(((W0, b0), (W1, b1), (W2, b2))):
        ps = _agg_partials(hcur, src3, dst3, zrow).reshape(NC, N, HID)
        res = _layer(hcur, ps, degp, W, b.reshape(1, HID), last=(i == 2))
        hcur = res[0]
    hsum = res[1]

    y = _head(hsum, p_pos, p_neg, FC_w)
    return jnp.squeeze(y)


# speedup vs baseline: 3.3996x; 1.0664x over previous
import functools

import jax
import jax.numpy as jnp
from jax import lax
from jax.experimental import pallas as pl
from jax.experimental.pallas import tpu as pltpu
from jax.experimental.pallas import tpu_sc as plsc

N = 10000
E = 320000
HID = 128
NPROT = 4

NC = 2
NS = 16
NW = NC * NS
CHK = 80
NCH = (E // NW) // CHK
DEGR = N
ZOFF = 624
ZLEN = 640

ROWS = 1000
GRID = N // ROWS

def _mesh():
    return plsc.VectorSubcoreMesh(core_axis_name="c", subcore_axis_name="s",
                                  num_cores=NC, num_subcores=NS)


def _deg_body(dst3, ones, zrows, out, dst_v, ones_v, acc, sem):
    c = lax.axis_index("c")
    s = lax.axis_index("s")
    wid = c * NS + s
    pltpu.sync_copy(zrows, acc.at[pl.ds(s * ZOFF, ZLEN)])
    pltpu.sync_copy(ones, ones_v)
    pltpu.sync_copy(dst3.at[wid], dst_v)
    plsc.subcore_barrier()

    def chunk(k, carry):
        pltpu.sync_copy(ones_v, acc.at[dst_v.at[k]], add=True)
        return carry

    lax.fori_loop(0, NCH, chunk, 0)
    plsc.subcore_barrier()
    pltpu.sync_copy(acc.at[pl.ds(s * ZOFF, ZLEN)],
                    out.at[pl.ds(c * N + s * ZOFF, ZLEN)])


def _deg_partials(dst3, ones, zrows):
    return pl.kernel(
        _deg_body,
        jax.ShapeDtypeStruct((NC * N, 16), jnp.float32),
        mesh=_mesh(),
        scratch_types=[
            pltpu.VMEM((NCH, CHK), jnp.int32),
            pltpu.VMEM((CHK, 16), jnp.float32),
            pltpu.VMEM_SHARED((DEGR, 16), jnp.float32),
            pltpu.SemaphoreType.DMA,
        ],
    )(dst3, ones, zrows)


HA = 63
HB = NCH - HA


def _agg_body(h, srcA3, srcB3, dst3, zrows, out,
              src_v, dst_v, rows0, rows1, acc, sem0, sem1):
    c = lax.axis_index("c")
    s = lax.axis_index("s")
    wid = c * NS + s
    pltpu.sync_copy(zrows, acc.at[pl.ds(s * ZOFF, ZLEN)])
    pltpu.sync_copy(srcA3.at[wid], src_v)
    pltpu.sync_copy(dst3.at[wid], dst_v)
    plsc.subcore_barrier()

    bufs = (rows0, rows1)
    sems = (sem0, sem1)

    def fire(kl, b):
        return pltpu.async_copy(h.at[src_v.at[kl]], bufs[b], sems[b])

    def scat(kg, b):
        pltpu.sync_copy(bufs[b], acc.at[dst_v.at[kg]], add=True)

    def ring(k0, off, n):
        cps = [None] * n
        cps[0] = fire(k0, 0)
        if n > 1:
            cps[1] = fire(k0 + 1, 1)
        for i in range(n):
            b = i % 2
            cps[i].wait()
            scat(off + k0 + i, b)
            if i + 2 < n:
                cps[i + 2] = fire(k0 + i + 2, b)

    def oct_a(j, carry):
        ring(8 * j, 0, 8)
        return carry

    lax.fori_loop(0, 7, oct_a, 0)
    ring(56, 0, 7)
    pltpu.sync_copy(srcB3.at[wid], src_v.at[pl.ds(0, HB)])

    def oct_b(j, carry):
        ring(8 * j, HA, 8)
        return carry

    lax.fori_loop(0, 7, oct_b, 0)
    ring(56, HA, 6)

    plsc.subcore_barrier()
    pltpu.sync_copy(acc.at[pl.ds(s * ZOFF, ZLEN)],
                    out.at[pl.ds(c * N + s * ZOFF, ZLEN)])


def _agg_partials(h, src3, dst3, zrows):
    return pl.kernel(
        _agg_body,
        jax.ShapeDtypeStruct((NC * N, HID), jnp.float32),
        mesh=_mesh(),
        scratch_types=[
            pltpu.VMEM((HA, CHK), jnp.int32),
            pltpu.VMEM((NCH, CHK), jnp.int32),
            pltpu.VMEM((CHK, HID), jnp.float32),
            pltpu.VMEM((CHK, HID), jnp.float32),
            pltpu.VMEM_SHARED((N, HID), jnp.float32),
            pltpu.SemaphoreType.DMA,
            pltpu.SemaphoreType.DMA,
        ],
    )(h, src3[:, :HA], src3[:, HA:], dst3, zrows)


def _embed_body(x_ref, w_ref, b_ref, o_ref):
    o_ref[...] = lax.dot_general(
        x_ref[...], w_ref[...], (((1,), (1,)), ((), ())),
        preferred_element_type=jnp.float32) + b_ref[...]


def _embed(x, w, b2):
    return pl.pallas_call(
        _embed_body,
        grid=(GRID,),
        in_specs=[
            pl.BlockSpec((ROWS, HID), lambda i: (i, 0)),
            pl.BlockSpec((HID, HID), lambda i: (0, 0)),
            pl.BlockSpec((1, HID), lambda i: (0, 0)),
        ],
        out_specs=pl.BlockSpec((ROWS, HID), lambda i: (i, 0)),
        out_shape=jax.ShapeDtypeStruct((N, HID), jnp.float32),
    )(x, w, b2)


def _layer_body(last, x_ref, ps_ref, dp_ref, w_ref, b_ref, o_ref, *rest):
    x = x_ref[...]
    ps = ps_ref[0] + ps_ref[1]
    deg = dp_ref[0, :, 0:1] + dp_ref[1, :, 0:1]
    agg = ps * (1.0 / jnp.maximum(deg, 1.0))
    w = w_ref[...]
    z = lax.dot_general(x, w[:, :HID], (((1,), (1,)), ((), ())),
                        preferred_element_type=jnp.float32)
    z = z + lax.dot_general(agg, w[:, HID:], (((1,), (1,)), ((), ())),
                            preferred_element_type=jnp.float32)
    z = z + b_ref[...]
    nrm = jnp.sqrt(jnp.sum(z * z, axis=1, keepdims=True))
    z = z / jnp.maximum(nrm, 1e-12)
    o = x + jnp.maximum(z, 0.0)
    o_ref[...] = o
    if last:
        hsum_ref = rest[0]
        @pl.when(pl.program_id(0) == 0)
        def _():
            hsum_ref[...] = jnp.zeros_like(hsum_ref)
        hsum_ref[...] += jnp.sum(o, axis=0, keepdims=True)


def _layer(x, ps, dp, w, b2, last):
    out_shape = [jax.ShapeDtypeStruct((N, HID), jnp.float32)]
    out_specs = [pl.BlockSpec((ROWS, HID), lambda i: (i, 0))]
    if last:
        out_shape.append(jax.ShapeDtypeStruct((1, HID), jnp.float32))
        out_specs.append(pl.BlockSpec((1, HID), lambda i: (0, 0)))
    return pl.pallas_call(
        functools.partial(_layer_body, last),
        grid=(GRID,),
        in_specs=[
            pl.BlockSpec((ROWS, HID), lambda i: (i, 0)),
            pl.BlockSpec((NC, ROWS, HID), lambda i: (0, i, 0)),
            pl.BlockSpec((NC, ROWS, 16), lambda i: (0, i, 0)),
            pl.BlockSpec((HID, 2 * HID), lambda i: (0, 0)),
            pl.BlockSpec((1, HID), lambda i: (0, 0)),
        ],
        out_specs=out_specs,
        out_shape=out_shape,
    )(x, ps, dp, w, b2)


def _head_body(hsum_ref, pp_ref, pn_ref, fc_ref, o_ref):
    hg = hsum_ref[...] * (1.0 / N)
    dp = hg - pp_ref[...]
    dn = hg - pn_ref[...]
    dpos = jnp.sum(dp * dp, axis=1, keepdims=True)
    dneg = jnp.sum(dn * dn, axis=1, keepdims=True)
    spos = jnp.log((dpos + 1.0) / (dpos + 1e-12))
    sneg = jnp.log((dneg + 1.0) / (dneg + 1e-12))
    fc = fc_ref[...]
    y = lax.dot_general(fc[:, :NPROT], spos, (((1,), (0,)), ((), ())),
                        preferred_element_type=jnp.float32)
    y = y + lax.dot_general(fc[:, NPROT:], sneg, (((1,), (0,)), ((), ())),
                            preferred_element_type=jnp.float32)
    o_ref[...] = 1.0 / (1.0 + jnp.exp(-y))


def _head(hsum, pp, pn, fc):
    return pl.pallas_call(
        _head_body,
        out_shape=jax.ShapeDtypeStruct((1, 1), jnp.float32),
    )(hsum, pp, pn, fc)


def kernel(h, edge_index, e, W_embed, b_embed, W0, b0, W1, b1, W2, b2,
           p_pos, p_neg, FC_w):
    src3 = edge_index[0].reshape(NW, NCH, CHK)
    dst3 = edge_index[1].reshape(NW, NCH, CHK)
    ones = jnp.ones((CHK, 16), jnp.float32)
    zdeg = jnp.zeros((ZLEN, 16), jnp.float32)
    zrow = jnp.zeros((ZLEN, HID), jnp.float32)

    degp = _deg_partials(dst3, ones, zdeg).reshape(NC, N, 16)
    hcur = _embed(h, W_embed, b_embed.reshape(1, HID))

    for i, (W, b) in enumerate(((W0, b0), (W1, b1), (W2, b2))):
        ps = _agg_partials(hcur, src3, dst3, zrow).reshape(NC, N, HID)
        res = _layer(hcur, ps, degp, W, b.reshape(1, HID), last=(i == 2))
        hcur = res[0]
    hsum = res[1]

    y = _head(hsum, p_pos, p_neg, FC_w)
    return jnp.squeeze(y)
